# baseline (speedup 1.0000x reference)
import jax
import jax.numpy as jnp
from jax import lax
from jax.experimental import pallas as pl
from jax.experimental.pallas import tpu as pltpu
from jax.experimental.pallas import tpu_sc as plsc

N = 100000
E = 1600000
ER = E // 128
NT = 16
CH = 5
WCH = 200
NWC = N // WCH

_f32 = jnp.float32
_i32 = jnp.int32


def _agg_pass(table, src_slice, dst2, agg, bufs, s, nch, row0):
    sb, db, rw, T, G, S = bufs
    nk = (nch - s + NT - 1) // NT

    def stage(m, q):
        r0 = row0(m)
        pltpu.async_copy(src_slice(pl.ds(r0, CH)), sb[q], T[q])
        pltpu.async_copy(dst2.at[pl.ds(r0, CH)], db[q], T[q])

    def wait_stage(q):
        pltpu.make_async_copy(src_slice(pl.ds(0, CH)), sb[q], T[q]).wait()
        pltpu.make_async_copy(dst2.at[pl.ds(0, CH)], db[q], T[q]).wait()

    def fire_gathers(q):
        for j in range(CH):
            pltpu.async_copy(table.at[sb[q].at[j]], rw[q].at[j], G[q])

    def wait_gathers(q):
        for j in range(CH):
            pltpu.make_async_copy(table.at[sb[q].at[j]], rw[q].at[j],
                                  G[q]).wait()

    def fire_scatters(q):
        for j in range(CH):
            pltpu.async_copy(rw[q].at[j], agg.at[db[q].at[j]], S[q], add=True)

    def drain_scatters(q):
        for j in range(CH):
            pltpu.make_async_copy(rw[q].at[j], agg.at[db[q].at[j]],
                                  S[q]).wait()

    def body(m, q, guard_m):
        def inner():
            wait_stage(q)
            fire_gathers(q)

            @pl.when(m >= 1)
            def _d():
                drain_scatters(1 - q)

            @pl.when(m + 1 < nk)
            def _s():
                stage(m + 1, 1 - q)

            wait_gathers(q)
            fire_scatters(q)

        if guard_m:
            pl.when(m < nk)(inner)
        else:
            inner()

    stage(0, 0)

    @pl.loop(0, (nk + 1) // 2)
    def _pair(t):
        body(2 * t, 0, guard_m=False)
        body(2 * t + 1, 1, guard_m=True)

    @pl.when(nk % 2 == 1)
    def _d0():
        drain_scatters(0)

    @pl.when(nk % 2 == 0)
    def _d1():
        drain_scatters(1)


def _zero_and_barrier(s, zbuf, agg):
    @pl.loop(0, (NWC - s + NT - 1) // NT)
    def _z(k):
        pltpu.sync_copy(zbuf, agg.at[pl.ds((s + k * NT) * WCH, WCH)])
    plsc.subcore_barrier()


_SC_SCRATCH = [
    pltpu.VMEM((CH, 128), _i32),
    pltpu.VMEM((CH, 128), _i32),
    pltpu.VMEM((CH, 128), _i32),
    pltpu.VMEM((CH, 128), _i32),
    pltpu.VMEM((CH, 128, 16), _f32),
    pltpu.VMEM((CH, 128, 16), _f32),
    pltpu.SemaphoreType.DMA,
    pltpu.SemaphoreType.DMA,
    pltpu.SemaphoreType.DMA,
    pltpu.SemaphoreType.DMA,
    pltpu.SemaphoreType.DMA,
    pltpu.SemaphoreType.DMA,
    pltpu.VMEM((WCH, 16), _f32),
    pltpu.VMEM_SHARED((N, 16), _f32),
]

_SC_MESH = plsc.VectorSubcoreMesh(core_axis_name="c", subcore_axis_name="s")


def _sc_agg16_body(x16, src2, dst2, out, sb0, sb1, db0, db1, rw0, rw1,
                   T0, T1, G0, G1, S0, S1, zbuf, agg):
    c = lax.axis_index("c")
    s = lax.axis_index("s")
    bufs = ((sb0, sb1), (db0, db1), (rw0, rw1), (T0, T1), (G0, G1), (S0, S1))

    @pl.loop(0, WCH)
    def _zb(i):
        zbuf[i, :] = jnp.zeros((16,), _f32)

    _zero_and_barrier(s, zbuf, agg)

    nch = ER // 2 // CH
    row_lo = c * (ER // 2)
    _agg_pass(x16, lambda sl: src2.at[sl], dst2, agg, bufs, s, nch,
              lambda m: row_lo + (s + m * NT) * CH)
    plsc.subcore_barrier()

    @pl.loop(0, (NWC - s + NT - 1) // NT)
    def _wo(k):
        ci = s + k * NT
        pltpu.sync_copy(agg.at[pl.ds(ci * WCH, WCH)],
                        out.at[pl.ds(c * N + ci * WCH, WCH)])


def _sc_agg128_body(tbl8, idx8, dst2, out, sb0, sb1, db0, db1, rw0, rw1,
                    T0, T1, G0, G1, S0, S1, zbuf, agg):
    c = lax.axis_index("c")
    s = lax.axis_index("s")
    bufs = ((sb0, sb1), (db0, db1), (rw0, rw1), (T0, T1), (G0, G1), (S0, S1))

    @pl.loop(0, WCH)
    def _zb(i):
        zbuf[i, :] = jnp.zeros((16,), _f32)

    for p in range(4):
        slab = 2 * p + c
        _zero_and_barrier(s, zbuf, agg)
        _agg_pass(tbl8, lambda sl, slab=slab: idx8.at[slab, sl], dst2, agg,
                  bufs, s, ER // CH, lambda m: (s + m * NT) * CH)
        plsc.subcore_barrier()

        @pl.loop(0, (NWC - s + NT - 1) // NT)
        def _wo(k):
            ci = s + k * NT
            pltpu.sync_copy(agg.at[pl.ds(ci * WCH, WCH)],
                            out.at[pl.ds(ci * WCH, WCH), slab])


_SC_PARAMS = pltpu.CompilerParams(use_tc_tiling_on_sc=False)

_sc_agg16 = pl.kernel(
    _sc_agg16_body,
    out_type=jax.ShapeDtypeStruct((2 * N, 16), _f32),
    mesh=_SC_MESH,
    scratch_types=_SC_SCRATCH,
    compiler_params=_SC_PARAMS,
)

_sc_agg128 = pl.kernel(
    _sc_agg128_body,
    out_type=jax.ShapeDtypeStruct((N, 8, 16), _f32),
    mesh=_SC_MESH,
    scratch_types=_SC_SCRATCH,
    compiler_params=_SC_PARAMS,
)


BT = 2000


def _inv_cnt(part_ref):
    cnt = part_ref[0, :, 9:10] + part_ref[1, :, 9:10]
    return 1.0 / jnp.maximum(cnt, 1.0)


def _combine1_body(part_ref, r_ref, wl_ref, o_ref):
    agg = part_ref[0] + part_ref[1]
    mean = agg * _inv_cnt(part_ref)
    h = jnp.dot(mean, wl_ref[...], preferred_element_type=_f32) + r_ref[...]
    o_ref[...] = jnp.maximum(h, 0.0)


def _r_body(x_ref, w_ref, b_ref, o_ref):
    o_ref[...] = (jnp.dot(x_ref[...], w_ref[...], preferred_element_type=_f32)
                  + b_ref[...])


def _combine_mid_body(agg_ref, part_ref, r_ref, wl_ref, o_ref):
    mean = agg_ref[...] * _inv_cnt(part_ref)
    h = jnp.dot(mean, wl_ref[...], preferred_element_type=_f32) + r_ref[...]
    o_ref[...] = jnp.maximum(h, 0.0)


def _combine_head_body(agg_ref, part_ref, r_ref, wl_ref,
                       w1_ref, b1_ref, w2_ref, b2_ref, w3_ref, b3_ref, o_ref):
    mean = agg_ref[...] * _inv_cnt(part_ref)
    h = jnp.dot(mean, wl_ref[...], preferred_element_type=_f32) + r_ref[...]
    h = jnp.maximum(h, 0.0)
    u = jnp.maximum(jnp.dot(h, w1_ref[...], preferred_element_type=_f32)
                    + b1_ref[...], 0.0)
    v = jnp.maximum(jnp.dot(u, w2_ref[...], preferred_element_type=_f32)
                    + b2_ref[...], 0.0)
    z = jnp.dot(v, w3_ref[...], preferred_element_type=_f32) + b3_ref[...]
    z3 = z[:, :3]
    m = jnp.max(z3, axis=1, keepdims=True)
    ez = jnp.exp(z3 - m)
    o_ref[...] = z3 - m - jnp.log(jnp.sum(ez, axis=1, keepdims=True))


def _part_spec():
    return pl.BlockSpec((2, BT, 16), lambda i: (0, i, 0))


def _full(shape):
    return pl.BlockSpec(shape, lambda i: tuple(0 for _ in shape))


def _r_call(xin, wT, b):
    k = xin.shape[1]
    return pl.pallas_call(
        _r_body,
        grid=(N // BT,),
        in_specs=[pl.BlockSpec((BT, k), lambda i: (i, 0)),
                  _full((k, 128)), _full((1, 128))],
        out_specs=pl.BlockSpec((BT, 128), lambda i: (i, 0)),
        out_shape=jax.ShapeDtypeStruct((N, 128), _f32),
    )(xin, wT, b)


def _combine1(part3, r1, wlT):
    return pl.pallas_call(
        _combine1_body,
        grid=(N // BT,),
        in_specs=[_part_spec(), pl.BlockSpec((BT, 128), lambda i: (i, 0)),
                  _full((16, 128))],
        out_specs=pl.BlockSpec((BT, 128), lambda i: (i, 0)),
        out_shape=jax.ShapeDtypeStruct((N, 128), _f32),
    )(part3, r1, wlT)


def _combine_mid(agg, part3, r, wlT):
    return pl.pallas_call(
        _combine_mid_body,
        grid=(N // BT,),
        in_specs=[pl.BlockSpec((BT, 128), lambda i: (i, 0)), _part_spec(),
                  pl.BlockSpec((BT, 128), lambda i: (i, 0)),
                  _full((128, 128))],
        out_specs=pl.BlockSpec((BT, 128), lambda i: (i, 0)),
        out_shape=jax.ShapeDtypeStruct((N, 128), _f32),
    )(agg, part3, r, wlT)


def _combine_head(agg, part3, r, wlT, w1T, b1, w2T, b2, w3T, b3):
    return pl.pallas_call(
        _combine_head_body,
        grid=(N // BT,),
        in_specs=[pl.BlockSpec((BT, 128), lambda i: (i, 0)), _part_spec(),
                  pl.BlockSpec((BT, 128), lambda i: (i, 0)),
                  _full((128, 128)),
                  _full((128, 128)), _full((1, 128)),
                  _full((128, 64)), _full((1, 64)),
                  _full((64, 8)), _full((1, 8))],
        out_specs=pl.BlockSpec((BT, 3), lambda i: (i, 0)),
        out_shape=jax.ShapeDtypeStruct((N, 3), _f32),
    )(agg, part3, r, wlT, w1T, b1, w2T, b2, w3T, b3)


@jax.jit
def kernel(x, edge_index, Wl1, bl1, Wr1, Wl2, bl2, Wr2, Wl3, bl3, Wr3,
           W1, b1, W2, b2, W3, b3):
    src2 = edge_index[0].astype(_i32).reshape(ER, 128)
    dst2 = edge_index[1].astype(_i32).reshape(ER, 128)
    idx8 = src2[None] * 8 + jnp.arange(8, dtype=_i32)[:, None, None]
    x16 = jnp.concatenate(
        [x, jnp.ones((N, 1), _f32), jnp.zeros((N, 6), _f32)], axis=1)
    wl1T = jnp.pad(Wl1, ((0, 0), (0, 7))).T
    wr1T = jnp.pad(Wr1, ((0, 0), (0, 7))).T
    w3T = jnp.pad(W3, ((0, 5), (0, 0))).T
    b3p = jnp.pad(b3, (0, 5)).reshape(1, 8)

    r1 = _r_call(x16, wr1T, bl1.reshape(1, 128))
    part = _sc_agg16(x16, src2, dst2)
    part3 = part.reshape(2, N, 16)
    h1 = _combine1(part3, r1, wl1T)

    r2 = _r_call(h1, Wr2.T, bl2.reshape(1, 128))
    agg2 = _sc_agg128(h1.reshape(N * 8, 16), idx8, dst2).reshape(N, 128)
    h2 = _combine_mid(agg2, part3, r2, Wl2.T)

    r3 = _r_call(h2, Wr3.T, bl3.reshape(1, 128))
    agg3 = _sc_agg128(h2.reshape(N * 8, 16), idx8, dst2).reshape(N, 128)
    return _combine_head(agg3, part3, r3, Wl3.T,
                         W1.T, b1.reshape(1, 128), W2.T, b2.reshape(1, 64),
                         w3T, b3p)


# speedup vs baseline: 1.2500x; 1.2500x over previous
import jax
import jax.numpy as jnp
from jax import lax
from jax.experimental import pallas as pl
from jax.experimental.pallas import tpu as pltpu
from jax.experimental.pallas import tpu_sc as plsc

N = 100000
E = 1600000
ER = E // 128
NT = 16
CH = 5
WCH = 200
NWC = N // WCH

_f32 = jnp.float32
_i32 = jnp.int32


def _agg_pass(table, src_slice, dst2, agg, bufs, s, nch, row0):
    sb, db, rw, T, G, S = bufs
    nk = (nch - s + NT - 1) // NT

    def stage(m, q):
        r0 = row0(m)
        pltpu.async_copy(src_slice(pl.ds(r0, CH)), sb[q], T[q])
        pltpu.async_copy(dst2.at[pl.ds(r0, CH)], db[q], T[q])

    def wait_stage(q):
        pltpu.make_async_copy(src_slice(pl.ds(0, CH)), sb[q], T[q]).wait()
        pltpu.make_async_copy(dst2.at[pl.ds(0, CH)], db[q], T[q]).wait()

    def fire_gathers(q):
        for j in range(CH):
            pltpu.async_copy(table.at[sb[q].at[j]], rw[q].at[j], G[q])

    def wait_gathers(q):
        for j in range(CH):
            pltpu.make_async_copy(table.at[sb[q].at[j]], rw[q].at[j],
                                  G[q]).wait()

    def fire_scatters(q):
        for j in range(CH):
            pltpu.async_copy(rw[q].at[j], agg.at[db[q].at[j]], S[q], add=True)

    def drain_scatters(q):
        for j in range(CH):
            pltpu.make_async_copy(rw[q].at[j], agg.at[db[q].at[j]],
                                  S[q]).wait()

    def body(m, q, guard_m):
        def inner():
            wait_stage(q)
            fire_gathers(q)

            @pl.when(m >= 1)
            def _d():
                drain_scatters(1 - q)

            @pl.when(m + 1 < nk)
            def _s():
                stage(m + 1, 1 - q)

            wait_gathers(q)
            fire_scatters(q)

        if guard_m:
            pl.when(m < nk)(inner)
        else:
            inner()

    stage(0, 0)

    @pl.loop(0, (nk + 1) // 2)
    def _pair(t):
        body(2 * t, 0, guard_m=False)
        body(2 * t + 1, 1, guard_m=True)

    @pl.when(nk % 2 == 1)
    def _d0():
        drain_scatters(0)

    @pl.when(nk % 2 == 0)
    def _d1():
        drain_scatters(1)


def _zero_and_barrier(s, zbuf, agg):
    @pl.loop(0, (NWC - s + NT - 1) // NT)
    def _z(k):
        pltpu.sync_copy(zbuf, agg.at[pl.ds((s + k * NT) * WCH, WCH)])
    plsc.subcore_barrier()


_SC_SCRATCH = [
    pltpu.VMEM((CH, 128), _i32),
    pltpu.VMEM((CH, 128), _i32),
    pltpu.VMEM((CH, 128), _i32),
    pltpu.VMEM((CH, 128), _i32),
    pltpu.VMEM((CH, 128, 16), _f32),
    pltpu.VMEM((CH, 128, 16), _f32),
    pltpu.SemaphoreType.DMA,
    pltpu.SemaphoreType.DMA,
    pltpu.SemaphoreType.DMA,
    pltpu.SemaphoreType.DMA,
    pltpu.SemaphoreType.DMA,
    pltpu.SemaphoreType.DMA,
    pltpu.VMEM((WCH, 16), _f32),
    pltpu.VMEM_SHARED((N, 16), _f32),
]

_SC_MESH = plsc.VectorSubcoreMesh(core_axis_name="c", subcore_axis_name="s")


def _sc_agg16_body(x16, src2, dst2, out, sb0, sb1, db0, db1, rw0, rw1,
                   T0, T1, G0, G1, S0, S1, zbuf, agg):
    c = lax.axis_index("c")
    s = lax.axis_index("s")
    bufs = ((sb0, sb1), (db0, db1), (rw0, rw1), (T0, T1), (G0, G1), (S0, S1))

    @pl.loop(0, WCH)
    def _zb(i):
        zbuf[i, :] = jnp.zeros((16,), _f32)

    _zero_and_barrier(s, zbuf, agg)

    nch = ER // 2 // CH
    row_lo = c * (ER // 2)
    _agg_pass(x16, lambda sl: src2.at[sl], dst2, agg, bufs, s, nch,
              lambda m: row_lo + (s + m * NT) * CH)
    plsc.subcore_barrier()

    @pl.loop(0, (NWC - s + NT - 1) // NT)
    def _wo(k):
        ci = s + k * NT
        pltpu.sync_copy(agg.at[pl.ds(ci * WCH, WCH)],
                        out.at[pl.ds(c * N + ci * WCH, WCH)])


def _sc_agg128_body(tbl8, idx8, dst2, out, sb0, sb1, db0, db1, rw0, rw1,
                    T0, T1, G0, G1, S0, S1, zbuf, agg):
    c = lax.axis_index("c")
    s = lax.axis_index("s")
    bufs = ((sb0, sb1), (db0, db1), (rw0, rw1), (T0, T1), (G0, G1), (S0, S1))

    @pl.loop(0, WCH)
    def _zb(i):
        zbuf[i, :] = jnp.zeros((16,), _f32)

    for p in range(4):
        slab = 2 * p + c
        _zero_and_barrier(s, zbuf, agg)
        _agg_pass(tbl8, lambda sl, slab=slab: idx8.at[slab, sl], dst2, agg,
                  bufs, s, ER // CH, lambda m: (s + m * NT) * CH)
        plsc.subcore_barrier()

        @pl.loop(0, (NWC - s + NT - 1) // NT)
        def _wo(k):
            ci = s + k * NT
            pltpu.sync_copy(agg.at[pl.ds(ci * WCH, WCH)],
                            out.at[pl.ds(ci * WCH, WCH),
                                   pl.ds(slab * 16, 16)])


_SC_PARAMS = pltpu.CompilerParams(use_tc_tiling_on_sc=False)

_sc_agg16 = pl.kernel(
    _sc_agg16_body,
    out_type=jax.ShapeDtypeStruct((2 * N, 16), _f32),
    mesh=_SC_MESH,
    scratch_types=_SC_SCRATCH,
    compiler_params=_SC_PARAMS,
)

_sc_agg128 = pl.kernel(
    _sc_agg128_body,
    out_type=jax.ShapeDtypeStruct((N, 128), _f32),
    mesh=_SC_MESH,
    scratch_types=_SC_SCRATCH,
    compiler_params=_SC_PARAMS,
)


BT = 2000


def _inv_cnt(part_ref):
    cnt = part_ref[0, :, 9:10] + part_ref[1, :, 9:10]
    return 1.0 / jnp.maximum(cnt, 1.0)


def _combine1_body(part_ref, r_ref, wl_ref, o_ref):
    agg = part_ref[0] + part_ref[1]
    mean = agg * _inv_cnt(part_ref)
    h = jnp.dot(mean, wl_ref[...], preferred_element_type=_f32) + r_ref[...]
    o_ref[...] = jnp.maximum(h, 0.0)


def _r_body(x_ref, w_ref, b_ref, o_ref):
    o_ref[...] = (jnp.dot(x_ref[...], w_ref[...], preferred_element_type=_f32)
                  + b_ref[...])


def _combine_mid_body(agg_ref, part_ref, r_ref, wl_ref, o_ref):
    mean = agg_ref[...] * _inv_cnt(part_ref)
    h = jnp.dot(mean, wl_ref[...], preferred_element_type=_f32) + r_ref[...]
    o_ref[...] = jnp.maximum(h, 0.0)


def _combine_head_body(agg_ref, part_ref, r_ref, wl_ref,
                       w1_ref, b1_ref, w2_ref, b2_ref, w3_ref, b3_ref, o_ref):
    mean = agg_ref[...] * _inv_cnt(part_ref)
    h = jnp.dot(mean, wl_ref[...], preferred_element_type=_f32) + r_ref[...]
    h = jnp.maximum(h, 0.0)
    u = jnp.maximum(jnp.dot(h, w1_ref[...], preferred_element_type=_f32)
                    + b1_ref[...], 0.0)
    v = jnp.maximum(jnp.dot(u, w2_ref[...], preferred_element_type=_f32)
                    + b2_ref[...], 0.0)
    z = jnp.dot(v, w3_ref[...], preferred_element_type=_f32) + b3_ref[...]
    z3 = z[:, :3]
    m = jnp.max(z3, axis=1, keepdims=True)
    ez = jnp.exp(z3 - m)
    o_ref[...] = z3 - m - jnp.log(jnp.sum(ez, axis=1, keepdims=True))


def _part_spec():
    return pl.BlockSpec((2, BT, 16), lambda i: (0, i, 0))


def _full(shape):
    return pl.BlockSpec(shape, lambda i: tuple(0 for _ in shape))


def _r_call(xin, wT, b):
    k = xin.shape[1]
    return pl.pallas_call(
        _r_body,
        grid=(N // BT,),
        in_specs=[pl.BlockSpec((BT, k), lambda i: (i, 0)),
                  _full((k, 128)), _full((1, 128))],
        out_specs=pl.BlockSpec((BT, 128), lambda i: (i, 0)),
        out_shape=jax.ShapeDtypeStruct((N, 128), _f32),
    )(xin, wT, b)


def _combine1(part3, r1, wlT):
    return pl.pallas_call(
        _combine1_body,
        grid=(N // BT,),
        in_specs=[_part_spec(), pl.BlockSpec((BT, 128), lambda i: (i, 0)),
                  _full((16, 128))],
        out_specs=pl.BlockSpec((BT, 128), lambda i: (i, 0)),
        out_shape=jax.ShapeDtypeStruct((N, 128), _f32),
    )(part3, r1, wlT)


def _combine_mid(agg, part3, r, wlT):
    return pl.pallas_call(
        _combine_mid_body,
        grid=(N // BT,),
        in_specs=[pl.BlockSpec((BT, 128), lambda i: (i, 0)), _part_spec(),
                  pl.BlockSpec((BT, 128), lambda i: (i, 0)),
                  _full((128, 128))],
        out_specs=pl.BlockSpec((BT, 128), lambda i: (i, 0)),
        out_shape=jax.ShapeDtypeStruct((N, 128), _f32),
    )(agg, part3, r, wlT)


def _combine_head(agg, part3, r, wlT, w1T, b1, w2T, b2, w3T, b3):
    return pl.pallas_call(
        _combine_head_body,
        grid=(N // BT,),
        in_specs=[pl.BlockSpec((BT, 128), lambda i: (i, 0)), _part_spec(),
                  pl.BlockSpec((BT, 128), lambda i: (i, 0)),
                  _full((128, 128)),
                  _full((128, 128)), _full((1, 128)),
                  _full((128, 64)), _full((1, 64)),
                  _full((64, 8)), _full((1, 8))],
        out_specs=pl.BlockSpec((BT, 3), lambda i: (i, 0)),
        out_shape=jax.ShapeDtypeStruct((N, 3), _f32),
    )(agg, part3, r, wlT, w1T, b1, w2T, b2, w3T, b3)


@jax.jit
def kernel(x, edge_index, Wl1, bl1, Wr1, Wl2, bl2, Wr2, Wl3, bl3, Wr3,
           W1, b1, W2, b2, W3, b3):
    src2 = edge_index[0].astype(_i32).reshape(ER, 128)
    dst2 = edge_index[1].astype(_i32).reshape(ER, 128)
    idx8 = src2[None] * 8 + jnp.arange(8, dtype=_i32)[:, None, None]
    x16 = jnp.concatenate(
        [x, jnp.ones((N, 1), _f32), jnp.zeros((N, 6), _f32)], axis=1)
    wl1T = jnp.pad(Wl1, ((0, 0), (0, 7))).T
    wr1T = jnp.pad(Wr1, ((0, 0), (0, 7))).T
    w3T = jnp.pad(W3, ((0, 5), (0, 0))).T
    b3p = jnp.pad(b3, (0, 5)).reshape(1, 8)

    r1 = _r_call(x16, wr1T, bl1.reshape(1, 128))
    part = _sc_agg16(x16, src2, dst2)
    part3 = part.reshape(2, N, 16)
    h1 = _combine1(part3, r1, wl1T)

    r2 = _r_call(h1, Wr2.T, bl2.reshape(1, 128))
    agg2 = _sc_agg128(h1.reshape(N * 8, 16), idx8, dst2)
    h2 = _combine_mid(agg2, part3, r2, Wl2.T)

    r3 = _r_call(h2, Wr3.T, bl3.reshape(1, 128))
    agg3 = _sc_agg128(h2.reshape(N * 8, 16), idx8, dst2)
    return _combine_head(agg3, part3, r3, Wl3.T,
                         W1.T, b1.reshape(1, 128), W2.T, b2.reshape(1, 64),
                         w3T, b3p)


# speedup vs baseline: 1.2640x; 1.0112x over previous
import jax
import jax.numpy as jnp
from jax import lax
from jax.experimental import pallas as pl
from jax.experimental.pallas import tpu as pltpu
from jax.experimental.pallas import tpu_sc as plsc

N = 100000
E = 1600000
ER = E // 128
NT = 16
CH = 5
WCH = 400
NWC = N // WCH

_f32 = jnp.float32
_i32 = jnp.int32


def _agg_pass(table, src_slice, dst2, agg, bufs, s, nch, row0):
    sb, db, rw, T, G, S = bufs
    nk = (nch - s + NT - 1) // NT

    def stage(m, q):
        r0 = row0(m)
        pltpu.async_copy(src_slice(pl.ds(r0, CH)), sb[q], T[q])
        pltpu.async_copy(dst2.at[pl.ds(r0, CH)], db[q], T[q])

    def wait_stage(q):
        pltpu.make_async_copy(src_slice(pl.ds(0, CH)), sb[q], T[q]).wait()
        pltpu.make_async_copy(dst2.at[pl.ds(0, CH)], db[q], T[q]).wait()

    def fire_gathers(q):
        for j in range(CH):
            pltpu.async_copy(table.at[sb[q].at[j]], rw[q].at[j], G[q])

    def wait_gathers(q):
        for j in range(CH):
            pltpu.make_async_copy(table.at[sb[q].at[j]], rw[q].at[j],
                                  G[q]).wait()

    def fire_scatters(q):
        for j in range(CH):
            pltpu.async_copy(rw[q].at[j], agg.at[db[q].at[j]], S[q], add=True)

    def drain_scatters(q):
        for j in range(CH):
            pltpu.make_async_copy(rw[q].at[j], agg.at[db[q].at[j]],
                                  S[q]).wait()

    def body(m, q, guard_m):
        def inner():
            wait_stage(q)
            fire_gathers(q)

            @pl.when(m >= 1)
            def _d():
                drain_scatters(1 - q)

            @pl.when(m + 1 < nk)
            def _s():
                stage(m + 1, 1 - q)

            wait_gathers(q)
            fire_scatters(q)

        if guard_m:
            pl.when(m < nk)(inner)
        else:
            inner()

    stage(0, 0)

    @pl.loop(0, (nk + 1) // 2)
    def _pair(t):
        body(2 * t, 0, guard_m=False)
        body(2 * t + 1, 1, guard_m=True)

    @pl.when(nk % 2 == 1)
    def _d0():
        drain_scatters(0)

    @pl.when(nk % 2 == 0)
    def _d1():
        drain_scatters(1)


def _zero_and_barrier(s, zbuf, agg):
    @pl.loop(0, (NWC - s + NT - 1) // NT)
    def _z(k):
        pltpu.sync_copy(zbuf, agg.at[pl.ds((s + k * NT) * WCH, WCH)])
    plsc.subcore_barrier()


_SC_SCRATCH = [
    pltpu.VMEM((CH, 128), _i32),
    pltpu.VMEM((CH, 128), _i32),
    pltpu.VMEM((CH, 128), _i32),
    pltpu.VMEM((CH, 128), _i32),
    pltpu.VMEM((CH, 128, 16), _f32),
    pltpu.VMEM((CH, 128, 16), _f32),
    pltpu.SemaphoreType.DMA,
    pltpu.SemaphoreType.DMA,
    pltpu.SemaphoreType.DMA,
    pltpu.SemaphoreType.DMA,
    pltpu.SemaphoreType.DMA,
    pltpu.SemaphoreType.DMA,
    pltpu.VMEM((WCH, 16), _f32),
    pltpu.VMEM_SHARED((N, 16), _f32),
]

_SC_MESH = plsc.VectorSubcoreMesh(core_axis_name="c", subcore_axis_name="s")


def _sc_agg16_body(x16, src2, dst2, out, sb0, sb1, db0, db1, rw0, rw1,
                   T0, T1, G0, G1, S0, S1, zbuf, agg):
    c = lax.axis_index("c")
    s = lax.axis_index("s")
    bufs = ((sb0, sb1), (db0, db1), (rw0, rw1), (T0, T1), (G0, G1), (S0, S1))

    @pl.loop(0, WCH)
    def _zb(i):
        zbuf[i, :] = jnp.zeros((16,), _f32)

    _zero_and_barrier(s, zbuf, agg)

    nch = ER // 2 // CH
    row_lo = c * (ER // 2)
    _agg_pass(x16, lambda sl: src2.at[sl], dst2, agg, bufs, s, nch,
              lambda m: row_lo + (s + m * NT) * CH)
    plsc.subcore_barrier()

    @pl.loop(0, (NWC - s + NT - 1) // NT)
    def _wo(k):
        ci = s + k * NT
        pltpu.sync_copy(agg.at[pl.ds(ci * WCH, WCH)],
                        out.at[pl.ds(c * N + ci * WCH, WCH)])


def _sc_agg128_body(tbl8, idx8, dst2, out, sb0, sb1, db0, db1, rw0, rw1,
                    T0, T1, G0, G1, S0, S1, zbuf, agg):
    c = lax.axis_index("c")
    s = lax.axis_index("s")
    bufs = ((sb0, sb1), (db0, db1), (rw0, rw1), (T0, T1), (G0, G1), (S0, S1))

    @pl.loop(0, WCH)
    def _zb(i):
        zbuf[i, :] = jnp.zeros((16,), _f32)

    for p in range(4):
        slab = 2 * p + c
        _zero_and_barrier(s, zbuf, agg)
        _agg_pass(tbl8, lambda sl, slab=slab: idx8.at[slab, sl], dst2, agg,
                  bufs, s, ER // CH, lambda m: (s + m * NT) * CH)
        plsc.subcore_barrier()

        @pl.loop(0, (NWC - s + NT - 1) // NT)
        def _wo(k):
            ci = s + k * NT
            pltpu.sync_copy(agg.at[pl.ds(ci * WCH, WCH)],
                            out.at[pl.ds(ci * WCH, WCH),
                                   pl.ds(slab * 16, 16)])


_SC_PARAMS = pltpu.CompilerParams(use_tc_tiling_on_sc=False,
                                  disable_bounds_checks=True)

_sc_agg16 = pl.kernel(
    _sc_agg16_body,
    out_type=jax.ShapeDtypeStruct((2 * N, 16), _f32),
    mesh=_SC_MESH,
    scratch_types=_SC_SCRATCH,
    compiler_params=_SC_PARAMS,
)

_sc_agg128 = pl.kernel(
    _sc_agg128_body,
    out_type=jax.ShapeDtypeStruct((N, 128), _f32),
    mesh=_SC_MESH,
    scratch_types=_SC_SCRATCH,
    compiler_params=_SC_PARAMS,
)


BT = 2000


def _inv_cnt(part_ref):
    cnt = part_ref[0, :, 9:10] + part_ref[1, :, 9:10]
    return 1.0 / jnp.maximum(cnt, 1.0)


def _combine1_body(part_ref, x_ref, wl_ref, bl_ref, wr_ref, o_ref):
    agg = part_ref[0] + part_ref[1]
    mean = agg * _inv_cnt(part_ref)
    h = jnp.dot(mean, wl_ref[...], preferred_element_type=_f32) + bl_ref[...]
    h = h + jnp.dot(x_ref[...], wr_ref[...], preferred_element_type=_f32)
    o_ref[...] = jnp.maximum(h, 0.0)


def _combine_mid_body(agg_ref, part_ref, hp_ref, wl_ref, bl_ref, wr_ref,
                      o_ref):
    mean = agg_ref[...] * _inv_cnt(part_ref)
    h = jnp.dot(mean, wl_ref[...], preferred_element_type=_f32) + bl_ref[...]
    h = h + jnp.dot(hp_ref[...], wr_ref[...], preferred_element_type=_f32)
    o_ref[...] = jnp.maximum(h, 0.0)


def _combine_head_body(agg_ref, part_ref, hp_ref, wl_ref, bl_ref, wr_ref,
                       w1_ref, b1_ref, w2_ref, b2_ref, w3_ref, b3_ref, o_ref):
    mean = agg_ref[...] * _inv_cnt(part_ref)
    h = jnp.dot(mean, wl_ref[...], preferred_element_type=_f32) + bl_ref[...]
    h = h + jnp.dot(hp_ref[...], wr_ref[...], preferred_element_type=_f32)
    h = jnp.maximum(h, 0.0)
    u = jnp.maximum(jnp.dot(h, w1_ref[...], preferred_element_type=_f32)
                    + b1_ref[...], 0.0)
    v = jnp.maximum(jnp.dot(u, w2_ref[...], preferred_element_type=_f32)
                    + b2_ref[...], 0.0)
    z = jnp.dot(v, w3_ref[...], preferred_element_type=_f32) + b3_ref[...]
    z3 = z[:, :3]
    m = jnp.max(z3, axis=1, keepdims=True)
    ez = jnp.exp(z3 - m)
    o_ref[...] = z3 - m - jnp.log(jnp.sum(ez, axis=1, keepdims=True))


def _part_spec():
    return pl.BlockSpec((2, BT, 16), lambda i: (0, i, 0))


def _full(shape):
    return pl.BlockSpec(shape, lambda i: tuple(0 for _ in shape))


def _row_spec(k=128):
    return pl.BlockSpec((BT, k), lambda i: (i, 0))


def _combine1(part3, x16, wlT, bl, wrT):
    return pl.pallas_call(
        _combine1_body,
        grid=(N // BT,),
        in_specs=[_part_spec(), _row_spec(16),
                  _full((16, 128)), _full((1, 128)), _full((16, 128))],
        out_specs=_row_spec(),
        out_shape=jax.ShapeDtypeStruct((N, 128), _f32),
    )(part3, x16, wlT, bl, wrT)


def _combine_mid(agg, part3, hp, wlT, bl, wrT):
    return pl.pallas_call(
        _combine_mid_body,
        grid=(N // BT,),
        in_specs=[_row_spec(), _part_spec(), _row_spec(),
                  _full((128, 128)), _full((1, 128)), _full((128, 128))],
        out_specs=_row_spec(),
        out_shape=jax.ShapeDtypeStruct((N, 128), _f32),
    )(agg, part3, hp, wlT, bl, wrT)


def _combine_head(agg, part3, hp, wlT, bl, wrT, w1T, b1, w2T, b2, w3T, b3):
    return pl.pallas_call(
        _combine_head_body,
        grid=(N // BT,),
        in_specs=[_row_spec(), _part_spec(), _row_spec(),
                  _full((128, 128)), _full((1, 128)), _full((128, 128)),
                  _full((128, 128)), _full((1, 128)),
                  _full((128, 64)), _full((1, 64)),
                  _full((64, 8)), _full((1, 8))],
        out_specs=pl.BlockSpec((BT, 3), lambda i: (i, 0)),
        out_shape=jax.ShapeDtypeStruct((N, 3), _f32),
    )(agg, part3, hp, wlT, bl, wrT, w1T, b1, w2T, b2, w3T, b3)


@jax.jit
def kernel(x, edge_index, Wl1, bl1, Wr1, Wl2, bl2, Wr2, Wl3, bl3, Wr3,
           W1, b1, W2, b2, W3, b3):
    src2 = edge_index[0].astype(_i32).reshape(ER, 128)
    dst2 = edge_index[1].astype(_i32).reshape(ER, 128)
    idx8 = src2[None] * 8 + jnp.arange(8, dtype=_i32)[:, None, None]
    x16 = jnp.concatenate(
        [x, jnp.ones((N, 1), _f32), jnp.zeros((N, 6), _f32)], axis=1)
    wl1T = jnp.pad(Wl1, ((0, 0), (0, 7))).T
    wr1T = jnp.pad(Wr1, ((0, 0), (0, 7))).T
    w3T = jnp.pad(W3, ((0, 5), (0, 0))).T
    b3p = jnp.pad(b3, (0, 5)).reshape(1, 8)

    part = _sc_agg16(x16, src2, dst2)
    part3 = part.reshape(2, N, 16)
    h1 = _combine1(part3, x16, wl1T, bl1.reshape(1, 128), wr1T)

    agg2 = _sc_agg128(h1.reshape(N * 8, 16), idx8, dst2)
    h2 = _combine_mid(agg2, part3, h1, Wl2.T, bl2.reshape(1, 128), Wr2.T)

    agg3 = _sc_agg128(h2.reshape(N * 8, 16), idx8, dst2)
    return _combine_head(agg3, part3, h2, Wl3.T, bl3.reshape(1, 128), Wr3.T,
                         W1.T, b1.reshape(1, 128), W2.T, b2.reshape(1, 64),
                         w3T, b3p)


# speedup vs baseline: 1.2651x; 1.0009x over previous
import jax
import jax.numpy as jnp
from jax import lax
from jax.experimental import pallas as pl
from jax.experimental.pallas import tpu as pltpu
from jax.experimental.pallas import tpu_sc as plsc

N = 100000
E = 1600000
ER = E // 128
NT = 16
CH = 5
WCH = 400
NWC = N // WCH

_f32 = jnp.float32
_i32 = jnp.int32


def _agg_pass(table, src_slice, dst2, agg, bufs, s, nch, row0):
    sb, db, rw, T, G, S = bufs
    nk = (nch - s + NT - 1) // NT

    def stage(m, q):
        r0 = row0(m)
        pltpu.async_copy(src_slice(pl.ds(r0, CH)), sb[q], T[q])
        pltpu.async_copy(dst2.at[pl.ds(r0, CH)], db[q], T[q])

    def wait_stage(q):
        pltpu.make_async_copy(src_slice(pl.ds(0, CH)), sb[q], T[q]).wait()
        pltpu.make_async_copy(dst2.at[pl.ds(0, CH)], db[q], T[q]).wait()

    def fire_gathers(q):
        for j in range(CH):
            pltpu.async_copy(table.at[sb[q].at[j]], rw[q].at[j], G[q])

    def wait_gathers(q):
        for j in range(CH):
            pltpu.make_async_copy(table.at[sb[q].at[j]], rw[q].at[j],
                                  G[q]).wait()

    def fire_scatters(q):
        for j in range(CH):
            pltpu.async_copy(rw[q].at[j], agg.at[db[q].at[j]], S[q], add=True)

    def drain_scatters(q):
        for j in range(CH):
            pltpu.make_async_copy(rw[q].at[j], agg.at[db[q].at[j]],
                                  S[q]).wait()

    def body(m, q, guard_m):
        def inner():
            wait_stage(q)
            fire_gathers(q)

            @pl.when(m >= 1)
            def _d():
                drain_scatters(1 - q)

            @pl.when(m + 1 < nk)
            def _s():
                stage(m + 1, 1 - q)

            wait_gathers(q)
            fire_scatters(q)

        if guard_m:
            pl.when(m < nk)(inner)
        else:
            inner()

    stage(0, 0)

    @pl.loop(0, (nk + 1) // 2)
    def _pair(t):
        body(2 * t, 0, guard_m=False)
        body(2 * t + 1, 1, guard_m=True)

    @pl.when(nk % 2 == 1)
    def _d0():
        drain_scatters(0)

    @pl.when(nk % 2 == 0)
    def _d1():
        drain_scatters(1)


def _zero_and_barrier(s, zbuf, agg):
    @pl.loop(0, (NWC - s + NT - 1) // NT)
    def _z(k):
        pltpu.sync_copy(zbuf, agg.at[pl.ds((s + k * NT) * WCH, WCH)])
    plsc.subcore_barrier()


_SC_SCRATCH = [
    pltpu.VMEM((CH, 128), _i32),
    pltpu.VMEM((CH, 128), _i32),
    pltpu.VMEM((CH, 128), _i32),
    pltpu.VMEM((CH, 128), _i32),
    pltpu.VMEM((CH, 128, 16), _f32),
    pltpu.VMEM((CH, 128, 16), _f32),
    pltpu.SemaphoreType.DMA,
    pltpu.SemaphoreType.DMA,
    pltpu.SemaphoreType.DMA,
    pltpu.SemaphoreType.DMA,
    pltpu.SemaphoreType.DMA,
    pltpu.SemaphoreType.DMA,
    pltpu.VMEM((WCH, 16), _f32),
    pltpu.VMEM_SHARED((N, 16), _f32),
]

_SC_MESH = plsc.VectorSubcoreMesh(core_axis_name="c", subcore_axis_name="s")


def _sc_agg16_body(x16, src2, dst2, out, sb0, sb1, db0, db1, rw0, rw1,
                   T0, T1, G0, G1, S0, S1, zbuf, agg):
    c = lax.axis_index("c")
    s = lax.axis_index("s")
    bufs = ((sb0, sb1), (db0, db1), (rw0, rw1), (T0, T1), (G0, G1), (S0, S1))

    @pl.loop(0, WCH)
    def _zb(i):
        zbuf[i, :] = jnp.zeros((16,), _f32)

    _zero_and_barrier(s, zbuf, agg)

    nch = ER // 2 // CH
    row_lo = c * (ER // 2)
    _agg_pass(x16, lambda sl: src2.at[sl], dst2, agg, bufs, s, nch,
              lambda m: row_lo + (s + m * NT) * CH)
    plsc.subcore_barrier()

    @pl.loop(0, (NWC - s + NT - 1) // NT)
    def _wo(k):
        ci = s + k * NT
        pltpu.sync_copy(agg.at[pl.ds(ci * WCH, WCH)],
                        out.at[pl.ds(c * N + ci * WCH, WCH)])


def _sc_agg128_body(tbl8, idx8, dst2, out, sb0, sb1, db0, db1, rw0, rw1,
                    T0, T1, G0, G1, S0, S1, zbuf, agg):
    c = lax.axis_index("c")
    s = lax.axis_index("s")
    bufs = ((sb0, sb1), (db0, db1), (rw0, rw1), (T0, T1), (G0, G1), (S0, S1))

    @pl.loop(0, WCH)
    def _zb(i):
        zbuf[i, :] = jnp.zeros((16,), _f32)

    @pl.loop(0, 4)
    def _pass(p):
        slab = 2 * p + c
        _zero_and_barrier(s, zbuf, agg)
        _agg_pass(tbl8, lambda sl: idx8.at[slab, sl], dst2, agg,
                  bufs, s, ER // CH, lambda m: (s + m * NT) * CH)
        plsc.subcore_barrier()

        @pl.loop(0, (NWC - s + NT - 1) // NT)
        def _wo(k):
            ci = s + k * NT
            pltpu.sync_copy(agg.at[pl.ds(ci * WCH, WCH)],
                            out.at[pl.ds(ci * WCH, WCH),
                                   pl.ds(slab * 16, 16)])


_SC_PARAMS = pltpu.CompilerParams(use_tc_tiling_on_sc=False,
                                  disable_bounds_checks=True)

_sc_agg16 = pl.kernel(
    _sc_agg16_body,
    out_type=jax.ShapeDtypeStruct((2 * N, 16), _f32),
    mesh=_SC_MESH,
    scratch_types=_SC_SCRATCH,
    compiler_params=_SC_PARAMS,
)

_sc_agg128 = pl.kernel(
    _sc_agg128_body,
    out_type=jax.ShapeDtypeStruct((N, 128), _f32),
    mesh=_SC_MESH,
    scratch_types=_SC_SCRATCH,
    compiler_params=_SC_PARAMS,
)


BT = 2000


def _inv_cnt(part_ref):
    cnt = part_ref[0, :, 9:10] + part_ref[1, :, 9:10]
    return 1.0 / jnp.maximum(cnt, 1.0)


def _combine1_body(part_ref, x_ref, wl_ref, bl_ref, wr_ref, o_ref):
    agg = part_ref[0] + part_ref[1]
    mean = agg * _inv_cnt(part_ref)
    h = jnp.dot(mean, wl_ref[...], preferred_element_type=_f32) + bl_ref[...]
    h = h + jnp.dot(x_ref[...], wr_ref[...], preferred_element_type=_f32)
    o_ref[...] = jnp.maximum(h, 0.0)


def _combine_mid_body(agg_ref, part_ref, hp_ref, wl_ref, bl_ref, wr_ref,
                      o_ref):
    mean = agg_ref[...] * _inv_cnt(part_ref)
    h = jnp.dot(mean, wl_ref[...], preferred_element_type=_f32) + bl_ref[...]
    h = h + jnp.dot(hp_ref[...], wr_ref[...], preferred_element_type=_f32)
    o_ref[...] = jnp.maximum(h, 0.0)


def _combine_head_body(agg_ref, part_ref, hp_ref, wl_ref, bl_ref, wr_ref,
                       w1_ref, b1_ref, w2_ref, b2_ref, w3_ref, b3_ref, o_ref):
    mean = agg_ref[...] * _inv_cnt(part_ref)
    h = jnp.dot(mean, wl_ref[...], preferred_element_type=_f32) + bl_ref[...]
    h = h + jnp.dot(hp_ref[...], wr_ref[...], preferred_element_type=_f32)
    h = jnp.maximum(h, 0.0)
    u = jnp.maximum(jnp.dot(h, w1_ref[...], preferred_element_type=_f32)
                    + b1_ref[...], 0.0)
    v = jnp.maximum(jnp.dot(u, w2_ref[...], preferred_element_type=_f32)
                    + b2_ref[...], 0.0)
    z = jnp.dot(v, w3_ref[...], preferred_element_type=_f32) + b3_ref[...]
    z3 = z[:, :3]
    m = jnp.max(z3, axis=1, keepdims=True)
    ez = jnp.exp(z3 - m)
    o_ref[...] = z3 - m - jnp.log(jnp.sum(ez, axis=1, keepdims=True))


def _part_spec():
    return pl.BlockSpec((2, BT, 16), lambda i: (0, i, 0))


def _full(shape):
    return pl.BlockSpec(shape, lambda i: tuple(0 for _ in shape))


def _row_spec(k=128):
    return pl.BlockSpec((BT, k), lambda i: (i, 0))


def _combine1(part3, x16, wlT, bl, wrT):
    return pl.pallas_call(
        _combine1_body,
        grid=(N // BT,),
        in_specs=[_part_spec(), _row_spec(16),
                  _full((16, 128)), _full((1, 128)), _full((16, 128))],
        out_specs=_row_spec(),
        out_shape=jax.ShapeDtypeStruct((N, 128), _f32),
    )(part3, x16, wlT, bl, wrT)


def _combine_mid(agg, part3, hp, wlT, bl, wrT):
    return pl.pallas_call(
        _combine_mid_body,
        grid=(N // BT,),
        in_specs=[_row_spec(), _part_spec(), _row_spec(),
                  _full((128, 128)), _full((1, 128)), _full((128, 128))],
        out_specs=_row_spec(),
        out_shape=jax.ShapeDtypeStruct((N, 128), _f32),
    )(agg, part3, hp, wlT, bl, wrT)


def _combine_head(agg, part3, hp, wlT, bl, wrT, w1T, b1, w2T, b2, w3T, b3):
    return pl.pallas_call(
        _combine_head_body,
        grid=(N // BT,),
        in_specs=[_row_spec(), _part_spec(), _row_spec(),
                  _full((128, 128)), _full((1, 128)), _full((128, 128)),
                  _full((128, 128)), _full((1, 128)),
                  _full((128, 64)), _full((1, 64)),
                  _full((64, 8)), _full((1, 8))],
        out_specs=pl.BlockSpec((BT, 3), lambda i: (i, 0)),
        out_shape=jax.ShapeDtypeStruct((N, 3), _f32),
    )(agg, part3, hp, wlT, bl, wrT, w1T, b1, w2T, b2, w3T, b3)


@jax.jit
def kernel(x, edge_index, Wl1, bl1, Wr1, Wl2, bl2, Wr2, Wl3, bl3, Wr3,
           W1, b1, W2, b2, W3, b3):
    src2 = edge_index[0].astype(_i32).reshape(ER, 128)
    dst2 = edge_index[1].astype(_i32).reshape(ER, 128)
    idx8 = src2[None] * 8 + jnp.arange(8, dtype=_i32)[:, None, None]
    x16 = jnp.concatenate(
        [x, jnp.ones((N, 1), _f32), jnp.zeros((N, 6), _f32)], axis=1)
    wl1T = jnp.pad(Wl1, ((0, 0), (0, 7))).T
    wr1T = jnp.pad(Wr1, ((0, 0), (0, 7))).T
    w3T = jnp.pad(W3, ((0, 5), (0, 0))).T
    b3p = jnp.pad(b3, (0, 5)).reshape(1, 8)

    part = _sc_agg16(x16, src2, dst2)
    part3 = part.reshape(2, N, 16)
    h1 = _combine1(part3, x16, wl1T, bl1.reshape(1, 128), wr1T)

    agg2 = _sc_agg128(h1.reshape(N * 8, 16), idx8, dst2)
    h2 = _combine_mid(agg2, part3, h1, Wl2.T, bl2.reshape(1, 128), Wr2.T)

    agg3 = _sc_agg128(h2.reshape(N * 8, 16), idx8, dst2)
    return _combine_head(agg3, part3, h2, Wl3.T, bl3.reshape(1, 128), Wr3.T,
                         W1.T, b1.reshape(1, 128), W2.T, b2.reshape(1, 64),
                         w3T, b3p)


# speedup vs baseline: 1.2735x; 1.0066x over previous
import jax
import jax.numpy as jnp
from jax import lax
from jax.experimental import pallas as pl
from jax.experimental.pallas import tpu as pltpu
from jax.experimental.pallas import tpu_sc as plsc

N = 100000
E = 1600000
ER = E // 128
NT = 16
CH = 5
WCH = 400
NWC = N // WCH

_f32 = jnp.float32
_i32 = jnp.int32


def _agg_pass(table, src_slice, dst2, agg, bufs, s, nch, row0):
    sb, db, rw, T, G, S = bufs
    nk = (nch - s + NT - 1) // NT

    def stage(m, q):
        r0 = row0(m)
        pltpu.async_copy(src_slice(pl.ds(r0, CH)), sb[q], T[q])
        pltpu.async_copy(dst2.at[pl.ds(r0, CH)], db[q], T[q])

    def wait_stage(q):
        pltpu.make_async_copy(src_slice(pl.ds(0, CH)), sb[q], T[q]).wait()
        pltpu.make_async_copy(dst2.at[pl.ds(0, CH)], db[q], T[q]).wait()

    def fire_gathers(q):
        for j in range(CH):
            pltpu.async_copy(table.at[sb[q].at[j]], rw[q].at[j], G[q])

    def wait_gathers(q):
        for j in range(CH):
            pltpu.make_async_copy(table.at[sb[q].at[j]], rw[q].at[j],
                                  G[q]).wait()

    def fire_scatters(q):
        for j in range(CH):
            pltpu.async_copy(rw[q].at[j], agg.at[db[q].at[j]], S[q], add=True)

    def drain_scatters(q):
        for j in range(CH):
            pltpu.make_async_copy(rw[q].at[j], agg.at[db[q].at[j]],
                                  S[q]).wait()

    def body(m, q, guard_m):
        def inner():
            wait_stage(q)
            fire_gathers(q)

            @pl.when(m >= 1)
            def _d():
                drain_scatters(1 - q)

            @pl.when(m + 1 < nk)
            def _s():
                stage(m + 1, 1 - q)

            wait_gathers(q)
            fire_scatters(q)

        if guard_m:
            pl.when(m < nk)(inner)
        else:
            inner()

    stage(0, 0)

    @pl.loop(0, (nk + 1) // 2)
    def _pair(t):
        body(2 * t, 0, guard_m=False)
        body(2 * t + 1, 1, guard_m=True)

    @pl.when(nk % 2 == 1)
    def _d0():
        drain_scatters(0)

    @pl.when(nk % 2 == 0)
    def _d1():
        drain_scatters(1)


def _zero_and_barrier(s, zbuf, agg):
    @pl.loop(0, (NWC - s + NT - 1) // NT)
    def _z(k):
        pltpu.sync_copy(zbuf, agg.at[pl.ds((s + k * NT) * WCH, WCH)])
    plsc.subcore_barrier()


_SC_SCRATCH = [
    pltpu.VMEM((CH, 128), _i32),
    pltpu.VMEM((CH, 128), _i32),
    pltpu.VMEM((CH, 128), _i32),
    pltpu.VMEM((CH, 128), _i32),
    pltpu.VMEM((CH, 128, 16), _f32),
    pltpu.VMEM((CH, 128, 16), _f32),
    pltpu.SemaphoreType.DMA,
    pltpu.SemaphoreType.DMA,
    pltpu.SemaphoreType.DMA,
    pltpu.SemaphoreType.DMA,
    pltpu.SemaphoreType.DMA,
    pltpu.SemaphoreType.DMA,
    pltpu.VMEM((WCH, 16), _f32),
    pltpu.VMEM_SHARED((N, 16), _f32),
]

_SC_MESH = plsc.VectorSubcoreMesh(core_axis_name="c", subcore_axis_name="s")


def _sc_agg16_body(x16, src2, dst2, out, sb0, sb1, db0, db1, rw0, rw1,
                   T0, T1, G0, G1, S0, S1, zbuf, agg):
    c = lax.axis_index("c")
    s = lax.axis_index("s")
    bufs = ((sb0, sb1), (db0, db1), (rw0, rw1), (T0, T1), (G0, G1), (S0, S1))

    @pl.loop(0, WCH)
    def _zb(i):
        zbuf[i, :] = jnp.zeros((16,), _f32)

    _zero_and_barrier(s, zbuf, agg)

    nch = ER // 2 // CH
    row_lo = c * (ER // 2)
    _agg_pass(x16, lambda sl: src2.at[sl], dst2, agg, bufs, s, nch,
              lambda m: row_lo + (s + m * NT) * CH)
    plsc.subcore_barrier()

    @pl.loop(0, (NWC - s + NT - 1) // NT)
    def _wo(k):
        ci = s + k * NT
        pltpu.sync_copy(agg.at[pl.ds(ci * WCH, WCH)],
                        out.at[pl.ds(c * N + ci * WCH, WCH)])


def _sc_agg128_body(tbl8, src8, dst2, out, sb0, sb1, db0, db1, rw0, rw1,
                    T0, T1, G0, G1, S0, S1, zbuf, agg):
    c = lax.axis_index("c")
    s = lax.axis_index("s")
    bufs = ((sb0, sb1), (db0, db1), (rw0, rw1), (T0, T1), (G0, G1), (S0, S1))

    @pl.loop(0, WCH)
    def _zb(i):
        zbuf[i, :] = jnp.zeros((16,), _f32)

    @pl.loop(0, 4)
    def _pass(p):
        slab = 2 * p + c
        _zero_and_barrier(s, zbuf, agg)
        _agg_pass(tbl8.at[pl.ds(slab, 8 * N - 7)],
                  lambda sl: src8.at[sl], dst2, agg,
                  bufs, s, ER // CH, lambda m: (s + m * NT) * CH)
        plsc.subcore_barrier()

        @pl.loop(0, (NWC - s + NT - 1) // NT)
        def _wo(k):
            ci = s + k * NT
            pltpu.sync_copy(agg.at[pl.ds(ci * WCH, WCH)],
                            out.at[pl.ds(ci * WCH, WCH),
                                   pl.ds(slab * 16, 16)])


_SC_PARAMS = pltpu.CompilerParams(use_tc_tiling_on_sc=False,
                                  disable_bounds_checks=True)

_sc_agg16 = pl.kernel(
    _sc_agg16_body,
    out_type=jax.ShapeDtypeStruct((2 * N, 16), _f32),
    mesh=_SC_MESH,
    scratch_types=_SC_SCRATCH,
    compiler_params=_SC_PARAMS,
)

_sc_agg128 = pl.kernel(
    _sc_agg128_body,
    out_type=jax.ShapeDtypeStruct((N, 128), _f32),
    mesh=_SC_MESH,
    scratch_types=_SC_SCRATCH,
    compiler_params=_SC_PARAMS,
)


BT = 2000


def _inv_cnt(part_ref):
    cnt = part_ref[0, :, 9:10] + part_ref[1, :, 9:10]
    return 1.0 / jnp.maximum(cnt, 1.0)


def _combine1_body(part_ref, x_ref, wl_ref, bl_ref, wr_ref, o_ref):
    agg = part_ref[0] + part_ref[1]
    mean = agg * _inv_cnt(part_ref)
    h = jnp.dot(mean, wl_ref[...], preferred_element_type=_f32) + bl_ref[...]
    h = h + jnp.dot(x_ref[...], wr_ref[...], preferred_element_type=_f32)
    o_ref[...] = jnp.maximum(h, 0.0)


def _combine_mid_body(agg_ref, part_ref, hp_ref, wl_ref, bl_ref, wr_ref,
                      o_ref):
    mean = agg_ref[...] * _inv_cnt(part_ref)
    h = jnp.dot(mean, wl_ref[...], preferred_element_type=_f32) + bl_ref[...]
    h = h + jnp.dot(hp_ref[...], wr_ref[...], preferred_element_type=_f32)
    o_ref[...] = jnp.maximum(h, 0.0)


def _combine_head_body(agg_ref, part_ref, hp_ref, wl_ref, bl_ref, wr_ref,
                       w1_ref, b1_ref, w2_ref, b2_ref, w3_ref, b3_ref, o_ref):
    mean = agg_ref[...] * _inv_cnt(part_ref)
    h = jnp.dot(mean, wl_ref[...], preferred_element_type=_f32) + bl_ref[...]
    h = h + jnp.dot(hp_ref[...], wr_ref[...], preferred_element_type=_f32)
    h = jnp.maximum(h, 0.0)
    u = jnp.maximum(jnp.dot(h, w1_ref[...], preferred_element_type=_f32)
                    + b1_ref[...], 0.0)
    v = jnp.maximum(jnp.dot(u, w2_ref[...], preferred_element_type=_f32)
                    + b2_ref[...], 0.0)
    z = jnp.dot(v, w3_ref[...], preferred_element_type=_f32) + b3_ref[...]
    z3 = z[:, :3]
    m = jnp.max(z3, axis=1, keepdims=True)
    ez = jnp.exp(z3 - m)
    o_ref[...] = z3 - m - jnp.log(jnp.sum(ez, axis=1, keepdims=True))


def _part_spec():
    return pl.BlockSpec((2, BT, 16), lambda i: (0, i, 0))


def _full(shape):
    return pl.BlockSpec(shape, lambda i: tuple(0 for _ in shape))


def _row_spec(k=128):
    return pl.BlockSpec((BT, k), lambda i: (i, 0))


def _combine1(part3, x16, wlT, bl, wrT):
    return pl.pallas_call(
        _combine1_body,
        grid=(N // BT,),
        in_specs=[_part_spec(), _row_spec(16),
                  _full((16, 128)), _full((1, 128)), _full((16, 128))],
        out_specs=_row_spec(),
        out_shape=jax.ShapeDtypeStruct((N, 128), _f32),
    )(part3, x16, wlT, bl, wrT)


def _combine_mid(agg, part3, hp, wlT, bl, wrT):
    return pl.pallas_call(
        _combine_mid_body,
        grid=(N // BT,),
        in_specs=[_row_spec(), _part_spec(), _row_spec(),
                  _full((128, 128)), _full((1, 128)), _full((128, 128))],
        out_specs=_row_spec(),
        out_shape=jax.ShapeDtypeStruct((N, 128), _f32),
    )(agg, part3, hp, wlT, bl, wrT)


def _combine_head(agg, part3, hp, wlT, bl, wrT, w1T, b1, w2T, b2, w3T, b3):
    return pl.pallas_call(
        _combine_head_body,
        grid=(N // BT,),
        in_specs=[_row_spec(), _part_spec(), _row_spec(),
                  _full((128, 128)), _full((1, 128)), _full((128, 128)),
                  _full((128, 128)), _full((1, 128)),
                  _full((128, 64)), _full((1, 64)),
                  _full((64, 8)), _full((1, 8))],
        out_specs=pl.BlockSpec((BT, 3), lambda i: (i, 0)),
        out_shape=jax.ShapeDtypeStruct((N, 3), _f32),
    )(agg, part3, hp, wlT, bl, wrT, w1T, b1, w2T, b2, w3T, b3)


@jax.jit
def kernel(x, edge_index, Wl1, bl1, Wr1, Wl2, bl2, Wr2, Wl3, bl3, Wr3,
           W1, b1, W2, b2, W3, b3):
    src2 = edge_index[0].astype(_i32).reshape(ER, 128)
    dst2 = edge_index[1].astype(_i32).reshape(ER, 128)
    src8 = src2 * 8
    x16 = jnp.concatenate(
        [x, jnp.ones((N, 1), _f32), jnp.zeros((N, 6), _f32)], axis=1)
    wl1T = jnp.pad(Wl1, ((0, 0), (0, 7))).T
    wr1T = jnp.pad(Wr1, ((0, 0), (0, 7))).T
    w3T = jnp.pad(W3, ((0, 5), (0, 0))).T
    b3p = jnp.pad(b3, (0, 5)).reshape(1, 8)

    part = _sc_agg16(x16, src2, dst2)
    part3 = part.reshape(2, N, 16)
    h1 = _combine1(part3, x16, wl1T, bl1.reshape(1, 128), wr1T)

    agg2 = _sc_agg128(h1.reshape(N * 8, 16), src8, dst2)
    h2 = _combine_mid(agg2, part3, h1, Wl2.T, bl2.reshape(1, 128), Wr2.T)

    agg3 = _sc_agg128(h2.reshape(N * 8, 16), src8, dst2)
    return _combine_head(agg3, part3, h2, Wl3.T, bl3.reshape(1, 128), Wr3.T,
                         W1.T, b1.reshape(1, 128), W2.T, b2.reshape(1, 64),
                         w3T, b3p)


# speedup vs baseline: 1.2898x; 1.0128x over previous
import jax
import jax.numpy as jnp
from jax import lax
from jax.experimental import pallas as pl
from jax.experimental.pallas import tpu as pltpu
from jax.experimental.pallas import tpu_sc as plsc

N = 100000
E = 1600000
ER = E // 128
NT = 16
CH = 5
WCH = 400
NWC = N // WCH

_f32 = jnp.float32
_i32 = jnp.int32


def _agg_pass(table, src_slice, dst2, agg, bufs, s, nch, row0):
    sb, db, rw, T, G, S = bufs
    nk = (nch - s + NT - 1) // NT

    def stage(m, q):
        r0 = row0(m)
        pltpu.async_copy(src_slice(pl.ds(r0, CH)), sb[q], T[q])
        pltpu.async_copy(dst2.at[pl.ds(r0, CH)], db[q], T[q])

    def wait_stage(q):
        pltpu.make_async_copy(src_slice(pl.ds(0, CH)), sb[q], T[q]).wait()
        pltpu.make_async_copy(dst2.at[pl.ds(0, CH)], db[q], T[q]).wait()

    def fire_gathers(q):
        for j in range(CH):
            pltpu.async_copy(table.at[sb[q].at[j]], rw[q].at[j], G[q])

    def wait_gathers(q):
        for j in range(CH):
            pltpu.make_async_copy(table.at[sb[q].at[j]], rw[q].at[j],
                                  G[q]).wait()

    def fire_scatters(q):
        for j in range(CH):
            pltpu.async_copy(rw[q].at[j], agg.at[db[q].at[j]], S[q], add=True)

    def drain_scatters(q):
        for j in range(CH):
            pltpu.make_async_copy(rw[q].at[j], agg.at[db[q].at[j]],
                                  S[q]).wait()

    def body(m, q, guard_m):
        def inner():
            wait_stage(q)
            fire_gathers(q)

            @pl.when(m >= 1)
            def _d():
                drain_scatters(1 - q)

            @pl.when(m + 1 < nk)
            def _s():
                stage(m + 1, 1 - q)

            wait_gathers(q)
            fire_scatters(q)

        if guard_m:
            pl.when(m < nk)(inner)
        else:
            inner()

    stage(0, 0)

    @pl.loop(0, (nk + 1) // 2)
    def _pair(t):
        body(2 * t, 0, guard_m=False)
        body(2 * t + 1, 1, guard_m=True)

    @pl.when(nk % 2 == 1)
    def _d0():
        drain_scatters(0)

    @pl.when(nk % 2 == 0)
    def _d1():
        drain_scatters(1)


def _zero_and_barrier(s, zbuf, agg):
    @pl.loop(0, (NWC - s + NT - 1) // NT)
    def _z(k):
        pltpu.sync_copy(zbuf, agg.at[pl.ds((s + k * NT) * WCH, WCH)])
    plsc.subcore_barrier()


_SC_SCRATCH = [
    pltpu.VMEM((CH, 128), _i32),
    pltpu.VMEM((CH, 128), _i32),
    pltpu.VMEM((CH, 128), _i32),
    pltpu.VMEM((CH, 128), _i32),
    pltpu.VMEM((CH, 128, 16), _f32),
    pltpu.VMEM((CH, 128, 16), _f32),
    pltpu.SemaphoreType.DMA,
    pltpu.SemaphoreType.DMA,
    pltpu.SemaphoreType.DMA,
    pltpu.SemaphoreType.DMA,
    pltpu.SemaphoreType.DMA,
    pltpu.SemaphoreType.DMA,
    pltpu.VMEM((WCH, 16), _f32),
    pltpu.VMEM_SHARED((N, 16), _f32),
]

_SC_MESH = plsc.VectorSubcoreMesh(core_axis_name="c", subcore_axis_name="s")


def _sc_agg16_body(x16, src2, dst2, out, sb0, sb1, db0, db1, rw0, rw1,
                   T0, T1, G0, G1, S0, S1, zbuf, agg):
    c = lax.axis_index("c")
    s = lax.axis_index("s")
    bufs = ((sb0, sb1), (db0, db1), (rw0, rw1), (T0, T1), (G0, G1), (S0, S1))

    @pl.loop(0, WCH)
    def _zb(i):
        zbuf[i, :] = jnp.zeros((16,), _f32)

    _zero_and_barrier(s, zbuf, agg)

    nch = ER // 2 // CH
    row_lo = c * (ER // 2)
    _agg_pass(x16, lambda sl: src2.at[sl], dst2, agg, bufs, s, nch,
              lambda m: row_lo + (s + m * NT) * CH)
    plsc.subcore_barrier()

    @pl.loop(0, (NWC - s + NT - 1) // NT)
    def _wo(k):
        ci = s + k * NT
        pltpu.sync_copy(agg.at[pl.ds(ci * WCH, WCH)],
                        out.at[pl.ds(c * N + ci * WCH, WCH)])


def _sc_agg128_body(tbl8, src8, dst2, out, sb0, sb1, db0, db1, rw0, rw1,
                    T0, T1, G0, G1, S0, S1, zbuf, agg):
    c = lax.axis_index("c")
    s = lax.axis_index("s")
    bufs = ((sb0, sb1), (db0, db1), (rw0, rw1), (T0, T1), (G0, G1), (S0, S1))

    @pl.loop(0, WCH)
    def _zb(i):
        zbuf[i, :] = jnp.zeros((16,), _f32)

    @pl.loop(0, 4)
    def _pass(p):
        slab = 2 * p + c
        _zero_and_barrier(s, zbuf, agg)
        _agg_pass(tbl8.at[pl.ds(slab, 8 * N - 7)],
                  lambda sl: src8.at[sl], dst2, agg,
                  bufs, s, ER // CH, lambda m: (s + m * NT) * CH)
        plsc.subcore_barrier()

        @pl.loop(0, (NWC - s + NT - 1) // NT)
        def _wo(k):
            ci = s + k * NT
            pltpu.sync_copy(agg.at[pl.ds(ci * WCH, WCH)],
                            out.at[pl.ds(ci * WCH, WCH),
                                   pl.ds(slab * 16, 16)])


_SC_PARAMS = pltpu.CompilerParams(use_tc_tiling_on_sc=False,
                                  disable_bounds_checks=True)

_sc_agg16 = pl.kernel(
    _sc_agg16_body,
    out_type=jax.ShapeDtypeStruct((2 * N, 16), _f32),
    mesh=_SC_MESH,
    scratch_types=_SC_SCRATCH,
    compiler_params=_SC_PARAMS,
)

_sc_agg128 = pl.kernel(
    _sc_agg128_body,
    out_type=jax.ShapeDtypeStruct((N, 128), _f32),
    mesh=_SC_MESH,
    scratch_types=_SC_SCRATCH,
    compiler_params=_SC_PARAMS,
)


BT = 4000


def _inv_cnt(part_ref):
    cnt = part_ref[0, :, 9:10] + part_ref[1, :, 9:10]
    return 1.0 / jnp.maximum(cnt, 1.0)


def _combine1_body(part_ref, x_ref, wl_ref, bl_ref, wr_ref, o_ref):
    agg = part_ref[0] + part_ref[1]
    mean = agg * _inv_cnt(part_ref)
    h = jnp.dot(mean, wl_ref[...], preferred_element_type=_f32) + bl_ref[...]
    h = h + jnp.dot(x_ref[...], wr_ref[...], preferred_element_type=_f32)
    o_ref[...] = jnp.maximum(h, 0.0)


def _combine_mid_body(agg_ref, part_ref, hp_ref, wl_ref, bl_ref, wr_ref,
                      o_ref):
    mean = agg_ref[...] * _inv_cnt(part_ref)
    h = jnp.dot(mean, wl_ref[...], preferred_element_type=_f32) + bl_ref[...]
    h = h + jnp.dot(hp_ref[...], wr_ref[...], preferred_element_type=_f32)
    o_ref[...] = jnp.maximum(h, 0.0)


def _combine_head_body(agg_ref, part_ref, hp_ref, wl_ref, bl_ref, wr_ref,
                       w1_ref, b1_ref, w2_ref, b2_ref, w3_ref, b3_ref, o_ref):
    mean = agg_ref[...] * _inv_cnt(part_ref)
    h = jnp.dot(mean, wl_ref[...], preferred_element_type=_f32) + bl_ref[...]
    h = h + jnp.dot(hp_ref[...], wr_ref[...], preferred_element_type=_f32)
    h = jnp.maximum(h, 0.0)
    u = jnp.maximum(jnp.dot(h, w1_ref[...], preferred_element_type=_f32)
                    + b1_ref[...], 0.0)
    v = jnp.maximum(jnp.dot(u, w2_ref[...], preferred_element_type=_f32)
                    + b2_ref[...], 0.0)
    z = jnp.dot(v, w3_ref[...], preferred_element_type=_f32) + b3_ref[...]
    z3 = z[:, :3]
    m = jnp.max(z3, axis=1, keepdims=True)
    ez = jnp.exp(z3 - m)
    o_ref[...] = z3 - m - jnp.log(jnp.sum(ez, axis=1, keepdims=True))


def _part_spec():
    return pl.BlockSpec((2, BT, 16), lambda i: (0, i, 0))


def _full(shape):
    return pl.BlockSpec(shape, lambda i: tuple(0 for _ in shape))


def _row_spec(k=128):
    return pl.BlockSpec((BT, k), lambda i: (i, 0))


def _combine1(part3, x16, wlT, bl, wrT):
    return pl.pallas_call(
        _combine1_body,
        grid=(N // BT,),
        in_specs=[_part_spec(), _row_spec(16),
                  _full((16, 128)), _full((1, 128)), _full((16, 128))],
        out_specs=_row_spec(),
        out_shape=jax.ShapeDtypeStruct((N, 128), _f32),
    )(part3, x16, wlT, bl, wrT)


def _combine_mid(agg, part3, hp, wlT, bl, wrT):
    return pl.pallas_call(
        _combine_mid_body,
        grid=(N // BT,),
        in_specs=[_row_spec(), _part_spec(), _row_spec(),
                  _full((128, 128)), _full((1, 128)), _full((128, 128))],
        out_specs=_row_spec(),
        out_shape=jax.ShapeDtypeStruct((N, 128), _f32),
    )(agg, part3, hp, wlT, bl, wrT)


def _combine_head(agg, part3, hp, wlT, bl, wrT, w1T, b1, w2T, b2, w3T, b3):
    return pl.pallas_call(
        _combine_head_body,
        grid=(N // BT,),
        in_specs=[_row_spec(), _part_spec(), _row_spec(),
                  _full((128, 128)), _full((1, 128)), _full((128, 128)),
                  _full((128, 128)), _full((1, 128)),
                  _full((128, 64)), _full((1, 64)),
                  _full((64, 8)), _full((1, 8))],
        out_specs=pl.BlockSpec((BT, 3), lambda i: (i, 0)),
        out_shape=jax.ShapeDtypeStruct((N, 3), _f32),
    )(agg, part3, hp, wlT, bl, wrT, w1T, b1, w2T, b2, w3T, b3)


@jax.jit
def kernel(x, edge_index, Wl1, bl1, Wr1, Wl2, bl2, Wr2, Wl3, bl3, Wr3,
           W1, b1, W2, b2, W3, b3):
    src2 = edge_index[0].astype(_i32).reshape(ER, 128)
    dst2 = edge_index[1].astype(_i32).reshape(ER, 128)
    src8 = src2 * 8
    x16 = jnp.concatenate(
        [x, jnp.ones((N, 1), _f32), jnp.zeros((N, 6), _f32)], axis=1)
    wl1T = jnp.pad(Wl1, ((0, 0), (0, 7))).T
    wr1T = jnp.pad(Wr1, ((0, 0), (0, 7))).T
    w3T = jnp.pad(W3, ((0, 5), (0, 0))).T
    b3p = jnp.pad(b3, (0, 5)).reshape(1, 8)

    part = _sc_agg16(x16, src2, dst2)
    part3 = part.reshape(2, N, 16)
    h1 = _combine1(part3, x16, wl1T, bl1.reshape(1, 128), wr1T)

    agg2 = _sc_agg128(h1.reshape(N * 8, 16), src8, dst2)
    h2 = _combine_mid(agg2, part3, h1, Wl2.T, bl2.reshape(1, 128), Wr2.T)

    agg3 = _sc_agg128(h2.reshape(N * 8, 16), src8, dst2)
    return _combine_head(agg3, part3, h2, Wl3.T, bl3.reshape(1, 128), Wr3.T,
                         W1.T, b1.reshape(1, 128), W2.T, b2.reshape(1, 64),
                         w3T, b3p)
